# baseline (speedup 1.0000x reference)
import functools
import math

import jax
import jax.numpy as jnp
from jax import lax
from jax.experimental import pallas as pl
from jax.experimental.pallas import tpu as pltpu
from jax.experimental.pallas import tpu_sc as plsc

_S = 128.0
_M = 0.1
_COS_M = math.cos(_M)
_SSIN_M = _S * math.sin(_M)


def _rsqrt(v, newton=3):
    i = lax.bitcast_convert_type(v, jnp.int32)
    y = lax.bitcast_convert_type(jnp.int32(0x5F3759DF) - (i >> 1), jnp.float32)
    for _ in range(newton):
        y = y * (1.5 - 0.5 * v * y * y)
    return y


def _log(z):
    i = lax.bitcast_convert_type(z, jnp.int32)
    e = (i >> 23) - 127
    m = lax.bitcast_convert_type((i & 0x7FFFFF) | 0x3F800000, jnp.float32)
    big = m > 1.4142135
    m = jnp.where(big, m * 0.5, m)
    e = e + jnp.where(big, 1, 0)
    u = (m - 1.0) / (m + 1.0)
    u2 = u * u
    p = u * (2.0 + u2 * (2.0 / 3.0 + u2 * (0.4 + u2 * (2.0 / 7.0))))
    return e.astype(jnp.float32) * 0.6931471805599453 + p


def _body(x_hbm, lab_hbm, w_hbm, loss_hbm, out_hbm,
          xbuf, lbuf, wbuf, obuf, lossbuf, smembuf, sem0, sem1, sem2):
    s = lax.axis_index("s")
    lanes = lax.broadcasted_iota(jnp.int32, (16,), 0)

    i0 = s * 16

    @pl.when(s < 8)
    def _():
        pltpu.async_copy(x_hbm.at[pl.ds(i0, 16)], xbuf, sem0)
        pltpu.async_copy(lab_hbm.at[pl.ds(i0, 16)], lbuf, sem1)
        pltpu.async_copy(w_hbm, wbuf, sem2)

    @pl.when(s == 0)
    def _():
        smembuf[0] = 0

    plsc.subcore_barrier()

    @pl.when(s < 8)
    def _():
        pltpu.make_async_copy(w_hbm, wbuf, sem2).wait()

        cl = jnp.minimum(lanes, 9)
        zeros_i = jnp.zeros((16,), jnp.int32)
        w0 = plsc.load_gather(wbuf, [zeros_i, cl])
        w1 = plsc.load_gather(wbuf, [zeros_i + 1, cl])
        g = _S * _rsqrt(jnp.maximum(w0 * w0 + w1 * w1, 1e-24))
        ws0 = w0 * g
        ws1 = w1 * g
        w0s = []
        w1s = []
        dnums = lax.GatherDimensionNumbers(
            offset_dims=(), collapsed_slice_dims=(0,), start_index_map=(0,))
        for j in range(10):
            jf = jnp.full((16, 1), j, jnp.int32)
            w0s.append(lax.gather(ws0, jf, dnums, (1,),
                                  mode=lax.GatherScatterMode.PROMISE_IN_BOUNDS))
            w1s.append(lax.gather(ws1, jf, dnums, (1,),
                                  mode=lax.GatherScatterMode.PROMISE_IN_BOUNDS))

        pltpu.make_async_copy(x_hbm.at[pl.ds(i0, 16)], xbuf, sem0).wait()
        pltpu.make_async_copy(lab_hbm.at[pl.ds(i0, 16)], lbuf, sem1).wait()
        a0 = plsc.load_gather(xbuf, [lanes, zeros_i])
        b0 = plsc.load_gather(xbuf, [lanes, zeros_i + 1])
        lab = lbuf[...]
        f = _rsqrt(jnp.maximum(a0 * a0 + b0 * b0, 1e-24))
        a = a0 * f
        b = b0 * f

        outs = []
        mx = jnp.full((16,), -3.0e38, jnp.float32)
        sc_lab = jnp.zeros((16,), jnp.float32)
        for j in range(10):
            oj = a * w0s[j] + b * w1s[j]
            plsc.store_scatter(obuf, [lanes, zeros_i + j], oj)
            outs.append(oj)
            mx = jnp.maximum(mx, oj)
            sc_lab = sc_lab + jnp.where(lab == j, oj, 0.0)

        c_lab = sc_lab * (1.0 / _S)
        sv = jnp.maximum(1.0 - c_lab * c_lab, 0.0)
        sine = sv * _rsqrt(jnp.maximum(sv, 1e-30), newton=2)
        phi_s = sc_lab * _COS_M - _SSIN_M * sine
        plsc.store_scatter(obuf, [lanes, lab], phi_s)
        mx = jnp.maximum(mx, phi_s)

        z = jnp.zeros((16,), jnp.float32)
        for j in range(10):
            ej = jnp.where(lab == j, phi_s, outs[j])
            z = z + jnp.exp(ej - mx)
        li = mx + _log(z) - phi_s

        part = (jnp.sum(li) * 32768.0).astype(jnp.int32)
        plsc.fetch_and_add(smembuf.at[0], part, subcore_id=0)
        pltpu.sync_copy(obuf, out_hbm.at[pl.ds(i0, 16)])

    plsc.subcore_barrier()

    @pl.when(s == 0)
    def _():
        total = smembuf[0]
        loss = total.astype(jnp.float32) * (1.0 / (32768.0 * 128.0))
        lossbuf[...] = jnp.full((16,), loss)
        pltpu.sync_copy(lossbuf.at[pl.ds(0, 8)], loss_hbm)


@functools.cache
def _make_kernel():
    return functools.partial(
        pl.kernel,
        out_type=(
            jax.ShapeDtypeStruct((8,), jnp.float32),
            jax.ShapeDtypeStruct((128, 10), jnp.float32),
        ),
        mesh=plsc.VectorSubcoreMesh(
            core_axis_name="c", subcore_axis_name="s", num_cores=1, num_subcores=16
        ),
        compiler_params=pltpu.CompilerParams(needs_layout_passes=False, use_tc_tiling_on_sc=True),
        scratch_types=[
            pltpu.VMEM((16, 2), jnp.float32),
            pltpu.VMEM((16,), jnp.int32),
            pltpu.VMEM((2, 10), jnp.float32),
            pltpu.VMEM((16, 10), jnp.float32),
            pltpu.VMEM((16,), jnp.float32),
            pltpu.SMEM((1,), jnp.int32),
            pltpu.SemaphoreType.DMA,
            pltpu.SemaphoreType.DMA,
            pltpu.SemaphoreType.DMA,
        ],
    )(_body)


@jax.jit
def kernel(x, label, W):
    loss_vec, out = _make_kernel()(x, label, W)
    return (loss_vec[0], out)


# speedup vs baseline: 1.0212x; 1.0212x over previous
import functools
import math

import jax
import jax.numpy as jnp
from jax import lax
from jax.experimental import pallas as pl
from jax.experimental.pallas import tpu as pltpu
from jax.experimental.pallas import tpu_sc as plsc

_S = 128.0
_M = 0.1
_COS_M = math.cos(_M)
_SSIN_M = _S * math.sin(_M)


def _rsqrt(v, newton=3):
    i = lax.bitcast_convert_type(v, jnp.int32)
    y = lax.bitcast_convert_type(jnp.int32(0x5F3759DF) - (i >> 1), jnp.float32)
    for _ in range(newton):
        y = y * (1.5 - 0.5 * v * y * y)
    return y


def _log(z):
    i = lax.bitcast_convert_type(z, jnp.int32)
    e = (i >> 23) - 127
    m = lax.bitcast_convert_type((i & 0x7FFFFF) | 0x3F800000, jnp.float32)
    big = m > 1.4142135
    m = jnp.where(big, m * 0.5, m)
    e = e + jnp.where(big, 1, 0)
    u = (m - 1.0) / (m + 1.0)
    u2 = u * u
    p = u * (2.0 + u2 * (2.0 / 3.0 + u2 * (0.4 + u2 * (2.0 / 7.0))))
    return e.astype(jnp.float32) * 0.6931471805599453 + p


def _body(x_hbm, lab_hbm, w_hbm, loss_hbm, out_hbm,
          xbuf, lbuf, wbuf, obuf, lossbuf, smembuf, sem0, sem1, sem2):
    s = lax.axis_index("s")
    lanes = lax.broadcasted_iota(jnp.int32, (16,), 0)

    i0 = s * 16

    @pl.when(s < 8)
    def _():
        pltpu.async_copy(x_hbm.at[pl.ds(i0 * 2, 32)], xbuf, sem0)
        pltpu.async_copy(lab_hbm.at[pl.ds(i0, 16)], lbuf, sem1)
        pltpu.async_copy(w_hbm, wbuf, sem2)

    @pl.when(s == 0)
    def _():
        smembuf[0] = 0

    plsc.subcore_barrier()

    @pl.when(s < 8)
    def _():
        pltpu.make_async_copy(w_hbm, wbuf, sem2).wait()

        cl = jnp.minimum(lanes, 9)
        w0 = plsc.load_gather(wbuf, [cl])
        w1 = plsc.load_gather(wbuf, [cl + 10])
        g = _S * _rsqrt(jnp.maximum(w0 * w0 + w1 * w1, 1e-24))
        ws0 = w0 * g
        ws1 = w1 * g
        w0s = []
        w1s = []
        dnums = lax.GatherDimensionNumbers(
            offset_dims=(), collapsed_slice_dims=(0,), start_index_map=(0,))
        for j in range(10):
            jf = jnp.full((16, 1), j, jnp.int32)
            w0s.append(lax.gather(ws0, jf, dnums, (1,),
                                  mode=lax.GatherScatterMode.PROMISE_IN_BOUNDS))
            w1s.append(lax.gather(ws1, jf, dnums, (1,),
                                  mode=lax.GatherScatterMode.PROMISE_IN_BOUNDS))

        pltpu.make_async_copy(x_hbm.at[pl.ds(i0 * 2, 32)], xbuf, sem0).wait()
        pltpu.make_async_copy(lab_hbm.at[pl.ds(i0, 16)], lbuf, sem1).wait()
        a0 = plsc.load_gather(xbuf, [lanes * 2])
        b0 = plsc.load_gather(xbuf, [lanes * 2 + 1])
        lab = lbuf[...]
        f = _rsqrt(jnp.maximum(a0 * a0 + b0 * b0, 1e-24))
        a = a0 * f
        b = b0 * f

        idx10 = lanes * 10
        outs = []
        mx = jnp.full((16,), -3.0e38, jnp.float32)
        sc_lab = jnp.zeros((16,), jnp.float32)
        for j in range(10):
            oj = a * w0s[j] + b * w1s[j]
            plsc.store_scatter(obuf, [idx10 + j], oj)
            outs.append(oj)
            mx = jnp.maximum(mx, oj)
            sc_lab = sc_lab + jnp.where(lab == j, oj, 0.0)

        c_lab = sc_lab * (1.0 / _S)
        sv = jnp.maximum(1.0 - c_lab * c_lab, 0.0)
        sine = sv * _rsqrt(jnp.maximum(sv, 1e-30), newton=2)
        phi_s = sc_lab * _COS_M - _SSIN_M * sine
        plsc.store_scatter(obuf, [idx10 + lab], phi_s)
        pltpu.async_copy(obuf, out_hbm.at[pl.ds(i0 * 10, 160)], sem0)
        mx = jnp.maximum(mx, phi_s)

        z = jnp.zeros((16,), jnp.float32)
        for j in range(10):
            ej = jnp.where(lab == j, phi_s, outs[j])
            z = z + jnp.exp(ej - mx)
        li = mx + _log(z) - phi_s

        part = (jnp.sum(li) * 32768.0).astype(jnp.int32)
        plsc.fetch_and_add(smembuf.at[0], part, subcore_id=0)
        pltpu.make_async_copy(obuf, out_hbm.at[pl.ds(i0 * 10, 160)], sem0).wait()

    plsc.subcore_barrier()

    @pl.when(s == 0)
    def _():
        total = smembuf[0]
        loss = total.astype(jnp.float32) * (1.0 / (32768.0 * 128.0))
        lossbuf[...] = jnp.full((16,), loss)
        pltpu.sync_copy(lossbuf.at[pl.ds(0, 8)], loss_hbm)


@functools.cache
def _make_kernel():
    return functools.partial(
        pl.kernel,
        out_type=(
            jax.ShapeDtypeStruct((8,), jnp.float32),
            jax.ShapeDtypeStruct((1280,), jnp.float32),
        ),
        mesh=plsc.VectorSubcoreMesh(
            core_axis_name="c", subcore_axis_name="s", num_cores=1, num_subcores=16
        ),
        compiler_params=pltpu.CompilerParams(needs_layout_passes=False),
        scratch_types=[
            pltpu.VMEM((32,), jnp.float32),
            pltpu.VMEM((16,), jnp.int32),
            pltpu.VMEM((20,), jnp.float32),
            pltpu.VMEM((160,), jnp.float32),
            pltpu.VMEM((16,), jnp.float32),
            pltpu.SMEM((1,), jnp.int32),
            pltpu.SemaphoreType.DMA,
            pltpu.SemaphoreType.DMA,
            pltpu.SemaphoreType.DMA,
        ],
    )(_body)


@jax.jit
def kernel(x, label, W):
    loss_vec, out_flat = _make_kernel()(x.reshape(256), label, W.reshape(20))
    return (loss_vec[0], out_flat.reshape(128, 10))
